# baseline (speedup 1.0000x reference)
import functools

import jax
import jax.numpy as jnp
from jax.experimental import pallas as pl
from jax.experimental.pallas import tpu as pltpu

_TILE_S = 512


def _stream_dot_kernel(ids_ref, tbl_ref, o_ref, cast_ref, oh_ref, *,
                       n_levels, vocab, seq):
    k = pl.program_id(0)
    tok = jax.lax.broadcasted_iota(jnp.int32, (1, vocab), 1)

    for l in range(n_levels):
        @pl.when(k == l)
        def _stream_level(l=l):
            cast_ref[l * vocab:(l + 1) * vocab, :] = (
                tbl_ref[...].astype(jnp.bfloat16))
            ids = ids_ref[l, :]
            oh_ref[:, l * vocab:(l + 1) * vocab] = (
                ids[:, None] == tok).astype(jnp.bfloat16)

    n_steps = seq // (2 * _TILE_S)
    for m in range(n_steps):
        @pl.when(k == n_levels + m)
        def _dot_pair(m=m):
            base = 2 * m * _TILE_S
            o_ref[:_TILE_S, :] = jnp.dot(
                oh_ref[base:base + _TILE_S, :], cast_ref[...],
                preferred_element_type=jnp.float32)
            o_ref[_TILE_S:, :] = jnp.dot(
                oh_ref[base + _TILE_S:base + 2 * _TILE_S, :], cast_ref[...],
                preferred_element_type=jnp.float32)


@functools.partial(jax.jit, static_argnames=("vocab",))
def _embed_sum(idx, tbl, *, vocab):
    n_levels, seq = idx.shape
    _, d = tbl.shape

    n_steps = seq // (2 * _TILE_S)
    body = functools.partial(_stream_dot_kernel, n_levels=n_levels,
                             vocab=vocab, seq=seq)
    return pl.pallas_call(
        body,
        out_shape=jax.ShapeDtypeStruct((seq, d), jnp.float32),
        grid=(n_levels + n_steps,),
        in_specs=[
            pl.BlockSpec((n_levels, seq), lambda k: (0, 0)),
            pl.BlockSpec((vocab, d),
                         lambda k: (jnp.minimum(k, n_levels - 1), 0)),
        ],
        out_specs=pl.BlockSpec(
            (2 * _TILE_S, d), lambda k: (jnp.maximum(k - n_levels, 0), 0),
            pipeline_mode=pl.Buffered(buffer_count=1)),
        scratch_shapes=[
            pltpu.VMEM((n_levels * vocab, d), jnp.bfloat16),
            pltpu.VMEM((seq, n_levels * vocab), jnp.bfloat16),
        ],
        compiler_params=pltpu.CompilerParams(
            dimension_semantics=("arbitrary",),
            vmem_limit_bytes=64 * 2**20),
    )(idx, tbl)


def kernel(xi, tables):
    xi = jnp.asarray(xi)
    n_levels = xi.shape[-1] - 1
    idx = jnp.transpose(xi[:, :n_levels]).astype(jnp.int32)
    n_tbl, n_tok, d = tables.shape
    tbl = tables.reshape(n_tbl * n_tok, d)
    return _embed_sum(idx, tbl, vocab=n_tok)


# speedup vs baseline: 1.1096x; 1.1096x over previous
import functools

import jax
import jax.numpy as jnp
from jax.experimental import pallas as pl
from jax.experimental.pallas import tpu as pltpu


def _two_phase_kernel(ids_ref, tbl_ref, o_ref, cast_ref, *,
                      n_levels, n_tiles, vocab, tile_s):
    k = pl.program_id(0)

    @pl.when(k < n_levels)
    def _cast_level():
        off = pl.multiple_of(k * vocab, vocab)
        cast_ref[pl.ds(off, vocab), :] = tbl_ref[...].astype(jnp.bfloat16)

    for m in range(n_tiles):
        @pl.when(k == n_levels + m)
        def _dot_tile(m=m):
            base = m * tile_s
            tok = jax.lax.broadcasted_iota(jnp.int32, (1, vocab), 1)
            parts = []
            for l in range(n_levels):
                ids = ids_ref[l, base:base + tile_s]
                parts.append((ids[:, None] == tok).astype(jnp.bfloat16))
            onehot = jnp.concatenate(parts, axis=1)
            o_ref[...] = jnp.dot(onehot, cast_ref[...],
                                 preferred_element_type=jnp.float32)


@functools.partial(jax.jit, static_argnames=("vocab",))
def _embed_sum(idx, tbl, *, vocab):
    n_levels, seq = idx.shape
    _, d = tbl.shape
    n_tiles = 2
    tile_s = seq // n_tiles

    body = functools.partial(_two_phase_kernel, n_levels=n_levels,
                             n_tiles=n_tiles, vocab=vocab, tile_s=tile_s)
    return pl.pallas_call(
        body,
        out_shape=jax.ShapeDtypeStruct((seq, d), jnp.float32),
        grid=(n_levels + n_tiles,),
        in_specs=[
            pl.BlockSpec((n_levels, seq), lambda k: (0, 0)),
            pl.BlockSpec((vocab, d),
                         lambda k: (jnp.minimum(k, n_levels - 1), 0)),
        ],
        out_specs=pl.BlockSpec(
            (tile_s, d), lambda k: (jnp.maximum(k - n_levels, 0), 0)),
        scratch_shapes=[
            pltpu.VMEM((n_levels * vocab, d), jnp.bfloat16),
        ],
        compiler_params=pltpu.CompilerParams(
            dimension_semantics=("arbitrary",),
            vmem_limit_bytes=64 * 2**20),
    )(idx, tbl)


def kernel(xi, tables):
    xi = jnp.asarray(xi)
    n_levels = xi.shape[-1] - 1
    idx = jnp.transpose(xi[:, :n_levels]).astype(jnp.int32)
    n_tbl, n_tok, d = tables.shape
    tbl = tables.reshape(n_tbl * n_tok, d)
    return _embed_sum(idx, tbl, vocab=n_tok)
